# baseline (speedup 1.0000x reference)
import jax
import jax.numpy as jnp
from jax.experimental import pallas as pl


_TILE = 1024


def _vq_body(z_ref, cb_ref, out_ref):
    cb = cb_ref[...]
    norm = jnp.sqrt(jnp.sum(cb * cb, axis=1, keepdims=True))
    cbn = cb / (norm + 1e-12)
    zb = z_ref[...]
    znorm = jnp.sqrt(jnp.sum(zb * zb, axis=1, keepdims=True))
    zn = zb / (znorm + 1e-12)
    dist = jax.lax.dot_general(
        zn, cbn, dimension_numbers=(((1,), (1,)), ((), ())),
        preferred_element_type=jnp.float32)
    k = dist.shape[1]
    m = jnp.max(dist, axis=1, keepdims=True)
    iota = jax.lax.broadcasted_iota(jnp.int32, dist.shape, 1)
    ind = jnp.min(jnp.where(dist == m, iota, k), axis=1, keepdims=True)
    onehot = (iota == ind).astype(jnp.float32)
    out_ref[...] = jnp.dot(onehot, cbn, precision=jax.lax.Precision.HIGHEST,
                           preferred_element_type=jnp.float32)


def kernel(z, codebook):
    shape = z.shape
    d = shape[-1]
    flat = z.reshape(-1, d)
    n = flat.shape[0]
    out = pl.pallas_call(
        _vq_body,
        grid=(n // _TILE,),
        in_specs=[
            pl.BlockSpec((_TILE, d), lambda i: (i, 0)),
            pl.BlockSpec(codebook.shape, lambda i: (0, 0)),
        ],
        out_specs=pl.BlockSpec((_TILE, d), lambda i: (i, 0)),
        out_shape=jax.ShapeDtypeStruct((n, d), jnp.float32),
    )(flat, codebook)
    return out.reshape(shape)


# speedup vs baseline: 1.7287x; 1.7287x over previous
import jax
import jax.numpy as jnp
from jax.experimental import pallas as pl
from jax.experimental.pallas import tpu as pltpu


_TILE = 1024


def _vq_body(z_ref, cb_ref, out_ref):
    cb = cb_ref[...]
    norm = jnp.sqrt(jnp.sum(cb * cb, axis=1, keepdims=True))
    cbn = cb / (norm + 1e-12)
    zb = z_ref[...]
    znorm = jnp.sqrt(jnp.sum(zb * zb, axis=1, keepdims=True))
    zn = zb / (znorm + 1e-12)
    dist = jax.lax.dot_general(
        zn, cbn, dimension_numbers=(((1,), (1,)), ((), ())),
        preferred_element_type=jnp.float32)
    k = dist.shape[1]
    m = jnp.max(dist, axis=1, keepdims=True)
    iota = jax.lax.broadcasted_iota(jnp.int32, dist.shape, 1)
    ind = jnp.min(jnp.where(dist == m, iota, k), axis=1, keepdims=True)
    onehot = (iota == ind).astype(jnp.float32)
    out_ref[...] = jnp.dot(onehot, cbn, preferred_element_type=jnp.float32)


def kernel(z, codebook):
    shape = z.shape
    d = shape[-1]
    flat = z.reshape(-1, d)
    n = flat.shape[0]
    out = pl.pallas_call(
        _vq_body,
        grid=(n // _TILE,),
        in_specs=[
            pl.BlockSpec((_TILE, d), lambda i: (i, 0)),
            pl.BlockSpec(codebook.shape, lambda i: (0, 0)),
        ],
        out_specs=pl.BlockSpec((_TILE, d), lambda i: (i, 0)),
        out_shape=jax.ShapeDtypeStruct((n, d), jnp.float32),
        compiler_params=pltpu.CompilerParams(
            dimension_semantics=("parallel",)),
    )(flat, codebook)
    return out.reshape(shape)
